# Initial kernel scaffold; baseline (speedup 1.0000x reference)
import jax
import jax.numpy as jnp
from jax.experimental import pallas as pl


def kernel(x, edge_index, W, a_src, a_dst):
    raise NotImplementedError("write your pallas kernel here")


# speedup vs baseline: 30.2419x; 30.2419x over previous
import functools

import jax
import jax.numpy as jnp
from jax import lax
from jax.experimental import pallas as pl
from jax.experimental.pallas import tpu as pltpu
from jax.experimental.pallas import tpu_sc as plsc

N = 10000
E = 320000
D = 128
NCORE = 2
NSUB = 16
NW = NCORE * NSUB
NP = 10240
SLICE = NP // NSUB
EPW = E // NW
CH = 80
NCHUNK = EPW // CH
NB = 25
SB = NCHUNK // NB
NBLK = 2000
GRID = N // NBLK


def _prep_body(x_ref, w_ref, asrc_ref, adst_ref, wh_ref, s_ref):
    wh = jnp.dot(x_ref[...], w_ref[...], preferred_element_type=jnp.float32)
    wh_ref[...] = wh
    av = asrc_ref[...] + adst_ref[...]
    s_ref[:N, :] = jnp.dot(wh, av, preferred_element_type=jnp.float32)
    s_ref[N:, :] = jnp.zeros((NP - N, 1), jnp.float32)


def _edge_kernel_body(wh_hbm, s_hbm, row_hbm, col_hbm,
                      acc_out, den_out,
                      s_loc, den_loc, ridx, cidx, wstore, rows,
                      acc_sh, sem):
    cid = lax.axis_index("c")
    sid = lax.axis_index("s")
    wid = sid * NCORE + cid

    zero16 = jnp.zeros((16,), jnp.float32)

    @pl.loop(0, NP // 16)
    def _zden(i):
        den_loc[pl.ds(i * 16, 16)] = zero16

    pltpu.sync_copy(s_hbm, s_loc)

    @pl.loop(0, CH)
    def _zrows(e):
        for v in range(D // 16):
            rows[e, pl.ds(v * 16, 16)] = zero16

    @pl.loop(0, SLICE // CH)
    def _zacc(q):
        pltpu.sync_copy(rows, acc_sh.at[pl.ds(sid * SLICE + q * CH, CH)])

    plsc.subcore_barrier()

    @pl.loop(0, SB)
    def _sblock(b):
        pltpu.sync_copy(row_hbm.at[wid, b], ridx)
        pltpu.sync_copy(col_hbm.at[wid, b], cidx)

        @pl.loop(0, NB)
        def _chunk(g):
            cp = pltpu.async_copy(wh_hbm.at[ridx.at[g]], rows, sem)
            for j in range(CH // 16):
                rv = ridx[g, pl.ds(j * 16, 16)]
                cv = cidx[g, pl.ds(j * 16, 16)]
                sr = plsc.load_gather(s_loc, [rv])
                sc = plsc.load_gather(s_loc, [cv])
                w = jnp.exp(sr + sc)
                wstore[pl.ds(j * 16, 16)] = w
                plsc.addupdate_scatter(den_loc, [cv], w)
            cp.wait()

            @pl.loop(0, CH)
            def _scale(e):
                evec = jnp.full((16,), 0, jnp.int32) + e
                wspl = plsc.load_gather(wstore, [evec])
                for v in range(D // 16):
                    rows[e, pl.ds(v * 16, 16)] = (
                        rows[e, pl.ds(v * 16, 16)] * wspl)

            pltpu.sync_copy(rows, acc_sh.at[cidx.at[g]], add=True)

    plsc.subcore_barrier()

    pltpu.sync_copy(acc_sh.at[pl.ds(sid * SLICE, SLICE)],
                    acc_out.at[cid, pl.ds(sid * SLICE, SLICE)])
    pltpu.sync_copy(den_loc, den_out.at[cid, sid])


_edge_kernel = functools.partial(
    pl.kernel,
    out_type=[jax.ShapeDtypeStruct((NCORE, NP, D), jnp.float32),
              jax.ShapeDtypeStruct((NCORE, NSUB, NP), jnp.float32)],
    mesh=plsc.VectorSubcoreMesh(core_axis_name="c", subcore_axis_name="s"),
    compiler_params=pltpu.CompilerParams(needs_layout_passes=False),
    scratch_types=[
        pltpu.VMEM((NP,), jnp.float32),
        pltpu.VMEM((NP,), jnp.float32),
        pltpu.VMEM((NB, CH), jnp.int32),
        pltpu.VMEM((NB, CH), jnp.int32),
        pltpu.VMEM((CH,), jnp.float32),
        pltpu.VMEM((CH, D), jnp.float32),
        pltpu.VMEM_SHARED((NP, D), jnp.float32),
        pltpu.SemaphoreType.DMA,
    ],
)(_edge_kernel_body)


def _finish_body(acc_ref, den_ref, out_ref):
    h = acc_ref[0] + acc_ref[1]
    d = jnp.sum(den_ref[...], axis=(0, 2)) + 1e-16
    h = h / d[:, None]
    out_ref[...] = jnp.where(h > 0.0, h, jnp.exp(jnp.minimum(h, 0.0)) - 1.0)


@jax.jit
def kernel(x, edge_index, W, a_src, a_dst):
    wh, s = pl.pallas_call(
        _prep_body,
        out_shape=[jax.ShapeDtypeStruct((N, D), jnp.float32),
                   jax.ShapeDtypeStruct((NP, 1), jnp.float32)],
    )(x, W, a_src, a_dst)

    row = edge_index[0].reshape(NW, SB, NB, CH)
    col = edge_index[1].reshape(NW, SB, NB, CH)

    acc, den = _edge_kernel(wh, s[:, 0], row, col)
    den = den.transpose(0, 2, 1)

    out = pl.pallas_call(
        _finish_body,
        grid=(GRID,),
        in_specs=[
            pl.BlockSpec((NCORE, NBLK, D), lambda i: (0, i, 0)),
            pl.BlockSpec((NCORE, NBLK, NSUB), lambda i: (0, i, 0)),
        ],
        out_specs=pl.BlockSpec((NBLK, D), lambda i: (i, 0)),
        out_shape=jax.ShapeDtypeStruct((N, D), jnp.float32),
    )(acc, den)
    return out


# speedup vs baseline: 34.4121x; 1.1379x over previous
import functools

import jax
import jax.numpy as jnp
from jax import lax
from jax.experimental import pallas as pl
from jax.experimental.pallas import tpu as pltpu
from jax.experimental.pallas import tpu_sc as plsc

N = 10000
E = 320000
D = 128
NCORE = 2
NSUB = 16
NW = NCORE * NSUB
NP = 10240
SLICE = NP // NSUB
EPW = E // NW
CH = 80
NCHUNK = EPW // CH
NB = 25
SB = NCHUNK // NB
NBLK = 2000
GRID = N // NBLK


def _prep_body(x_ref, w_ref, asrc_ref, adst_ref, wh_ref, s_ref):
    wh = jnp.dot(x_ref[...], w_ref[...], preferred_element_type=jnp.float32)
    wh_ref[...] = wh
    av = asrc_ref[...] + adst_ref[...]
    s_ref[:N, :] = jnp.dot(wh, av, preferred_element_type=jnp.float32)
    s_ref[N:, :] = jnp.zeros((NP - N, 1), jnp.float32)


def _edge_kernel_body(wh_hbm, s_hbm, row_hbm, col_hbm,
                      acc_out, den_out,
                      s_loc, den_loc, ridx, cidx, wstore, rows,
                      wstore2, rows2, sem2,
                      acc_sh, sem):
    cid = lax.axis_index("c")
    sid = lax.axis_index("s")
    wid = sid * NCORE + cid

    zero16 = jnp.zeros((16,), jnp.float32)

    @pl.loop(0, N // 16)
    def _zden(i):
        den_loc[pl.ds(i * 16, 16)] = zero16

    pltpu.sync_copy(s_hbm, s_loc)

    @pl.loop(0, CH)
    def _zrows(e):
        for v in range(D // 16):
            rows[e, pl.ds(v * 16, 16)] = zero16

    @pl.loop(0, SLICE // CH)
    def _zacc(q):
        pltpu.sync_copy(rows, acc_sh.at[pl.ds(sid * SLICE + q * CH, CH)])

    plsc.subcore_barrier()

    def _wcompute(g, wst):
        for j in range(CH // 16):
            rv = ridx[g, pl.ds(j * 16, 16)]
            cv = cidx[g, pl.ds(j * 16, 16)]
            sr = plsc.load_gather(s_loc, [rv])
            sc = plsc.load_gather(s_loc, [cv])
            w = jnp.exp(sr + sc)
            wst[pl.ds(j * 16, 16)] = w
            plsc.addupdate_scatter(den_loc, [cv], w)

    def _wscale(rbuf, wst):
        @pl.loop(0, CH)
        def _scale(e):
            evec = jnp.full((16,), 0, jnp.int32) + e
            wspl = plsc.load_gather(wst, [evec])
            for v in range(D // 16):
                rbuf[e, pl.ds(v * 16, 16)] = (
                    rbuf[e, pl.ds(v * 16, 16)] * wspl)

    @pl.loop(0, SB)
    def _sblock(b):
        pltpu.sync_copy(row_hbm.at[wid, b], ridx)
        pltpu.sync_copy(col_hbm.at[wid, b], cidx)

        @pl.loop(0, NB // 2)
        def _chunk2(q):
            g0 = q * 2
            g1 = g0 + 1
            cp0 = pltpu.async_copy(wh_hbm.at[ridx.at[g0]], rows, sem)
            _wcompute(g0, wstore)
            cp1 = pltpu.async_copy(wh_hbm.at[ridx.at[g1]], rows2, sem2)
            _wcompute(g1, wstore2)
            cp0.wait()
            _wscale(rows, wstore)
            pltpu.sync_copy(rows, acc_sh.at[cidx.at[g0]], add=True)
            cp1.wait()
            _wscale(rows2, wstore2)
            pltpu.sync_copy(rows2, acc_sh.at[cidx.at[g1]], add=True)

        gt = NB - 1
        cpt = pltpu.async_copy(wh_hbm.at[ridx.at[gt]], rows, sem)
        _wcompute(gt, wstore)
        cpt.wait()
        _wscale(rows, wstore)
        pltpu.sync_copy(rows, acc_sh.at[cidx.at[gt]], add=True)

    plsc.subcore_barrier()

    pltpu.sync_copy(acc_sh.at[pl.ds(sid * SLICE, SLICE)],
                    acc_out.at[cid, pl.ds(sid * SLICE, SLICE)])
    pltpu.sync_copy(den_loc, den_out.at[cid, sid])


_edge_kernel = functools.partial(
    pl.kernel,
    out_type=[jax.ShapeDtypeStruct((NCORE, NP, D), jnp.float32),
              jax.ShapeDtypeStruct((NCORE, NSUB, N), jnp.float32)],
    mesh=plsc.VectorSubcoreMesh(core_axis_name="c", subcore_axis_name="s"),
    compiler_params=pltpu.CompilerParams(needs_layout_passes=False),
    scratch_types=[
        pltpu.VMEM((N,), jnp.float32),
        pltpu.VMEM((N,), jnp.float32),
        pltpu.VMEM((NB, CH), jnp.int32),
        pltpu.VMEM((NB, CH), jnp.int32),
        pltpu.VMEM((CH,), jnp.float32),
        pltpu.VMEM((CH, D), jnp.float32),
        pltpu.VMEM((CH,), jnp.float32),
        pltpu.VMEM((CH, D), jnp.float32),
        pltpu.SemaphoreType.DMA,
        pltpu.VMEM_SHARED((NP, D), jnp.float32),
        pltpu.SemaphoreType.DMA,
    ],
)(_edge_kernel_body)


def _finish_body(acc_ref, den_ref, out_ref):
    h = acc_ref[0] + acc_ref[1]
    d = jnp.sum(den_ref[...], axis=(0, 2)) + 1e-16
    h = h / d[:, None]
    out_ref[...] = jnp.where(h > 0.0, h, jnp.exp(jnp.minimum(h, 0.0)) - 1.0)


@jax.jit
def kernel(x, edge_index, W, a_src, a_dst):
    wh, s = pl.pallas_call(
        _prep_body,
        out_shape=[jax.ShapeDtypeStruct((N, D), jnp.float32),
                   jax.ShapeDtypeStruct((NP, 1), jnp.float32)],
    )(x, W, a_src, a_dst)

    row = edge_index[0].reshape(NW, SB, NB, CH)
    col = edge_index[1].reshape(NW, SB, NB, CH)

    acc, den = _edge_kernel(wh, s[:N, 0], row, col)
    den = den.transpose(0, 2, 1)

    out = pl.pallas_call(
        _finish_body,
        grid=(GRID,),
        in_specs=[
            pl.BlockSpec((NCORE, NBLK, D), lambda i: (0, i, 0)),
            pl.BlockSpec((NCORE, NBLK, NSUB), lambda i: (0, i, 0)),
        ],
        out_specs=pl.BlockSpec((NBLK, D), lambda i: (i, 0)),
        out_shape=jax.ShapeDtypeStruct((N, D), jnp.float32),
    )(acc, den)
    return out


# speedup vs baseline: 37.1982x; 1.0810x over previous
import functools

import jax
import jax.numpy as jnp
from jax import lax
from jax.experimental import pallas as pl
from jax.experimental.pallas import tpu as pltpu
from jax.experimental.pallas import tpu_sc as plsc

N = 10000
E = 320000
D = 128
NCORE = 2
NSUB = 16
NW = NCORE * NSUB
NP = 10240
SLICE = NP // NSUB
EPW = E // NW
CH = 80
NCHUNK = EPW // CH
NB = 25
SB = NCHUNK // NB
NBLK = 2000
GRID = N // NBLK


def _prep_body(x_ref, w_ref, asrc_ref, adst_ref, wh_ref, s_ref):
    wh = jnp.dot(x_ref[...], w_ref[...], preferred_element_type=jnp.float32)
    wh_ref[...] = wh
    av = asrc_ref[...] + adst_ref[...]
    s_ref[:N, :] = jnp.dot(wh, av, preferred_element_type=jnp.float32)
    s_ref[N:, :] = jnp.zeros((NP - N, 1), jnp.float32)


def _edge_kernel_body(wh_hbm, s_hbm, row_hbm, col_hbm,
                      acc_out, den_out,
                      s_loc, den_loc, ridx, cidx, wstore, rows,
                      wstore2, rows2, sem2, sem3,
                      acc_sh, sem):
    cid = lax.axis_index("c")
    sid = lax.axis_index("s")
    wid = sid * NCORE + cid

    zero16 = jnp.zeros((16,), jnp.float32)

    @pl.loop(0, N // 16)
    def _zden(i):
        den_loc[pl.ds(i * 16, 16)] = zero16

    pltpu.sync_copy(s_hbm, s_loc)

    @pl.loop(0, CH)
    def _zrows(e):
        for v in range(D // 16):
            rows[e, pl.ds(v * 16, 16)] = zero16

    @pl.loop(0, SLICE // CH)
    def _zacc(q):
        pltpu.sync_copy(rows, acc_sh.at[pl.ds(sid * SLICE + q * CH, CH)])

    plsc.subcore_barrier()

    def _wcompute(g, wst):
        for j in range(CH // 16):
            rv = ridx[g, pl.ds(j * 16, 16)]
            cv = cidx[g, pl.ds(j * 16, 16)]
            sr = plsc.load_gather(s_loc, [rv])
            sc = plsc.load_gather(s_loc, [cv])
            w = jnp.exp(sr + sc)
            wst[pl.ds(j * 16, 16)] = w
            plsc.addupdate_scatter(den_loc, [cv], w)

    def _wscale(rbuf, wst):
        @pl.loop(0, CH)
        def _scale(e):
            evec = jnp.full((16,), 0, jnp.int32) + e
            wspl = plsc.load_gather(wst, [evec])
            for v in range(D // 16):
                rbuf[e, pl.ds(v * 16, 16)] = (
                    rbuf[e, pl.ds(v * 16, 16)] * wspl)

    @pl.loop(0, SB)
    def _sblock(b):
        pltpu.sync_copy(row_hbm.at[wid, b], ridx)
        pltpu.sync_copy(col_hbm.at[wid, b], cidx)

        @pl.loop(0, NB // 2)
        def _chunk2(q):
            g0 = q * 2
            g1 = g0 + 1
            cp0 = pltpu.async_copy(wh_hbm.at[ridx.at[g0]], rows, sem)
            _wcompute(g0, wstore)
            cp1 = pltpu.async_copy(wh_hbm.at[ridx.at[g1]], rows2, sem2)
            _wcompute(g1, wstore2)
            cp0.wait()
            _wscale(rows, wstore)
            sc0 = pltpu.async_copy(rows, acc_sh.at[cidx.at[g0]], sem3,
                                   add=True)
            cp1.wait()
            _wscale(rows2, wstore2)
            pltpu.sync_copy(rows2, acc_sh.at[cidx.at[g1]], add=True)
            sc0.wait()

        gt = NB - 1
        cpt = pltpu.async_copy(wh_hbm.at[ridx.at[gt]], rows, sem)
        _wcompute(gt, wstore)
        cpt.wait()
        _wscale(rows, wstore)
        pltpu.sync_copy(rows, acc_sh.at[cidx.at[gt]], add=True)

    plsc.subcore_barrier()

    pltpu.sync_copy(acc_sh.at[pl.ds(sid * SLICE, SLICE)],
                    acc_out.at[cid, pl.ds(sid * SLICE, SLICE)])
    pltpu.sync_copy(den_loc, den_out.at[cid, sid])


_edge_kernel = functools.partial(
    pl.kernel,
    out_type=[jax.ShapeDtypeStruct((NCORE, NP, D), jnp.float32),
              jax.ShapeDtypeStruct((NCORE, NSUB, N), jnp.float32)],
    mesh=plsc.VectorSubcoreMesh(core_axis_name="c", subcore_axis_name="s"),
    compiler_params=pltpu.CompilerParams(needs_layout_passes=False),
    scratch_types=[
        pltpu.VMEM((N,), jnp.float32),
        pltpu.VMEM((N,), jnp.float32),
        pltpu.VMEM((NB, CH), jnp.int32),
        pltpu.VMEM((NB, CH), jnp.int32),
        pltpu.VMEM((CH,), jnp.float32),
        pltpu.VMEM((CH, D), jnp.float32),
        pltpu.VMEM((CH,), jnp.float32),
        pltpu.VMEM((CH, D), jnp.float32),
        pltpu.SemaphoreType.DMA,
        pltpu.SemaphoreType.DMA,
        pltpu.VMEM_SHARED((NP, D), jnp.float32),
        pltpu.SemaphoreType.DMA,
    ],
)(_edge_kernel_body)


def _finish_body(acc_ref, den_ref, out_ref):
    h = acc_ref[0] + acc_ref[1]
    d = jnp.sum(den_ref[...], axis=(0, 2)) + 1e-16
    h = h / d[:, None]
    out_ref[...] = jnp.where(h > 0.0, h, jnp.exp(jnp.minimum(h, 0.0)) - 1.0)


@jax.jit
def kernel(x, edge_index, W, a_src, a_dst):
    wh, s = pl.pallas_call(
        _prep_body,
        out_shape=[jax.ShapeDtypeStruct((N, D), jnp.float32),
                   jax.ShapeDtypeStruct((NP, 1), jnp.float32)],
    )(x, W, a_src, a_dst)

    row = edge_index[0].reshape(NW, SB, NB, CH)
    col = edge_index[1].reshape(NW, SB, NB, CH)

    acc, den = _edge_kernel(wh, s[:N, 0], row, col)
    den = den.transpose(0, 2, 1)

    out = pl.pallas_call(
        _finish_body,
        grid=(GRID,),
        in_specs=[
            pl.BlockSpec((NCORE, NBLK, D), lambda i: (0, i, 0)),
            pl.BlockSpec((NCORE, NBLK, NSUB), lambda i: (0, i, 0)),
        ],
        out_specs=pl.BlockSpec((NBLK, D), lambda i: (i, 0)),
        out_shape=jax.ShapeDtypeStruct((N, D), jnp.float32),
    )(acc, den)
    return out
